# Initial kernel scaffold; baseline (speedup 1.0000x reference)
import jax
import jax.numpy as jnp
from jax.experimental import pallas as pl


def kernel(log_x_recon, cf_log_x_recon):
    raise NotImplementedError("write your pallas kernel here")


# speedup vs baseline: 28.8490x; 28.8490x over previous
import functools

import jax
import jax.numpy as jnp
import numpy as np
from jax.experimental import pallas as pl

_GUIDANCE_SCALE = 3.0
_TRUNCATION_R = 0.86
_NEG_CLIP = -70.0
_INT32_MAX = 2147483647


def _sort_key(v):
    i = jax.lax.bitcast_convert_type(v, jnp.int32)
    return jnp.where(v == 0.0, np.int32(_INT32_MAX), ~i)


def _topr_kernel(lx_ref, cf_ref, out_ref, *, n_iters, lo_init):
    lx = lx_ref[...]
    cf = cf_ref[...]
    g = cf + _GUIDANCE_SCALE * (lx - cf)
    m = jnp.max(g, axis=1, keepdims=True)
    e = jnp.exp(g - m)
    s = jnp.sum(e, axis=1, keepdims=True)
    v = jnp.clip(g - (m + jnp.log(s)), _NEG_CLIP, 0.0)
    p = e * (1.0 / s)
    keys = _sort_key(v)

    lo0 = jnp.full(m.shape, lo_init, dtype=jnp.int32)
    hi0 = jnp.full(m.shape, _INT32_MAX, dtype=jnp.int32)

    def body(_, carry):
        lo, hi = carry
        mid = lo + jax.lax.shift_right_logical(hi - lo, 1)
        mass = jnp.sum(jnp.where(keys > mid, p, 0.0), axis=1, keepdims=True)
        pred = mass < _TRUNCATION_R
        return jnp.where(pred, lo, mid), jnp.where(pred, mid, hi)

    _, theta = jax.lax.fori_loop(0, n_iters, body, (lo0, hi0))
    out_ref[...] = jnp.where(keys >= theta, v, _NEG_CLIP)


@jax.jit
def kernel(log_x_recon, cf_log_x_recon):
    b_dim, k_dim, hw = log_x_recon.shape
    tile = hw
    for cand in (256, 128, 64, 32, 16, 8):
        if hw % cand == 0:
            tile = cand
            break
    lo_init = int(~np.float32(_NEG_CLIP).view(np.int32)) - 1
    span = int(_INT32_MAX) - lo_init
    n_iters = max(1, span.bit_length())

    grid = (b_dim, hw // tile)
    blk = pl.BlockSpec((1, k_dim, tile), lambda b, t: (b, 0, t))
    return pl.pallas_call(
        functools.partial(_topr_kernel, n_iters=n_iters, lo_init=lo_init),
        grid=grid,
        in_specs=[blk, blk],
        out_specs=blk,
        out_shape=jax.ShapeDtypeStruct(log_x_recon.shape, log_x_recon.dtype),
    )(log_x_recon, cf_log_x_recon)


# speedup vs baseline: 32.3779x; 1.1223x over previous
import functools

import jax
import jax.numpy as jnp
import numpy as np
from jax.experimental import pallas as pl

_GUIDANCE_SCALE = 3.0
_TRUNCATION_R = 0.86
_NEG_CLIP = -70.0


def _topr_kernel(lx_ref, cf_ref, out_ref, *, n_iters, lo_init, hi_init):
    lx = lx_ref[...]
    cf = cf_ref[...]
    g = cf + _GUIDANCE_SCALE * (lx - cf)
    m = jnp.max(g, axis=1, keepdims=True)
    e = jnp.exp(g - m)
    s = jnp.sum(e, axis=1, keepdims=True)
    rs = _TRUNCATION_R * s

    lo0 = jnp.full(m.shape, lo_init, dtype=jnp.int32)
    hi0 = jnp.full(m.shape, hi_init, dtype=jnp.int32)

    def body(_, carry):
        lo, hi = carry
        mid = lo + jax.lax.shift_right_logical(hi - lo, 1)
        t = jax.lax.bitcast_convert_type(mid, jnp.float32)
        mass = jnp.sum(jnp.where(e > t, e, 0.0), axis=1, keepdims=True)
        pred = mass < rs
        return jnp.where(pred, lo, mid), jnp.where(pred, mid, hi)

    lo_fin, _ = jax.lax.fori_loop(0, n_iters, body, (lo0, hi0))
    t_lo = jax.lax.bitcast_convert_type(lo_fin, jnp.float32)
    v = jnp.clip(g - (m + jnp.log(s)), _NEG_CLIP, 0.0)
    out_ref[...] = jnp.where(e > t_lo, v, _NEG_CLIP)


@jax.jit
def kernel(log_x_recon, cf_log_x_recon):
    b_dim, k_dim, hw = log_x_recon.shape
    tile = hw
    for cand in (256, 128, 64, 32, 16, 8):
        if hw % cand == 0:
            tile = cand
            break
    lo_init = int(np.float32(np.exp(np.float32(-11.0))).view(np.int32)) - 1
    hi_init = int(np.float32(1.0).view(np.int32))
    n_iters = max(1, (hi_init - lo_init).bit_length())

    grid = (b_dim, hw // tile)
    blk = pl.BlockSpec((1, k_dim, tile), lambda b, t: (b, 0, t))
    return pl.pallas_call(
        functools.partial(_topr_kernel, n_iters=n_iters,
                          lo_init=lo_init, hi_init=hi_init),
        grid=grid,
        in_specs=[blk, blk],
        out_specs=blk,
        out_shape=jax.ShapeDtypeStruct(log_x_recon.shape, log_x_recon.dtype),
    )(log_x_recon, cf_log_x_recon)


# speedup vs baseline: 36.7075x; 1.1337x over previous
import functools

import jax
import jax.numpy as jnp
import numpy as np
from jax.experimental import pallas as pl

_GUIDANCE_SCALE = 3.0
_TRUNCATION_R = 0.86
_NEG_CLIP = -70.0


_CHUNK = 64


def _masked_chunk_sum(e, t, k_dim):
    n_full = k_dim // _CHUNK
    ch = jax.lax.slice_in_dim(e, 0, _CHUNK, axis=1)
    acc = jnp.where(ch > t, ch, 0.0)
    for c in range(1, n_full):
        ch = jax.lax.slice_in_dim(e, c * _CHUNK, (c + 1) * _CHUNK, axis=1)
        acc = acc + jnp.where(ch > t, ch, 0.0)
    mass = jnp.sum(acc, axis=1, keepdims=True)
    if k_dim % _CHUNK:
        ch = jax.lax.slice_in_dim(e, n_full * _CHUNK, k_dim, axis=1)
        mass = mass + jnp.sum(jnp.where(ch > t, ch, 0.0), axis=1,
                              keepdims=True)
    return mass


def _chunk_sum(e, k_dim):
    n_full = k_dim // _CHUNK
    acc = jax.lax.slice_in_dim(e, 0, _CHUNK, axis=1)
    for c in range(1, n_full):
        acc = acc + jax.lax.slice_in_dim(e, c * _CHUNK, (c + 1) * _CHUNK,
                                         axis=1)
    s = jnp.sum(acc, axis=1, keepdims=True)
    if k_dim % _CHUNK:
        s = s + jnp.sum(jax.lax.slice_in_dim(e, n_full * _CHUNK, k_dim,
                                             axis=1), axis=1, keepdims=True)
    return s


def _topr_kernel(lx_ref, cf_ref, out_ref, *, n_iters, lo_init, hi_init):
    lx = lx_ref[...]
    cf = cf_ref[...]
    g = cf + _GUIDANCE_SCALE * (lx - cf)
    k_dim = g.shape[1]
    m = jnp.max(g, axis=1, keepdims=True)
    e = jnp.exp(g - m)
    s = _chunk_sum(e, k_dim)
    rs = _TRUNCATION_R * s

    lo0 = jnp.full(m.shape, lo_init, dtype=jnp.int32)
    hi0 = jnp.full(m.shape, hi_init, dtype=jnp.int32)

    def body(_, carry):
        lo, hi = carry
        mid = lo + jax.lax.shift_right_logical(hi - lo, 1)
        t = jax.lax.bitcast_convert_type(mid, jnp.float32)
        mass = _masked_chunk_sum(e, t, k_dim)
        pred = mass < rs
        return jnp.where(pred, lo, mid), jnp.where(pred, mid, hi)

    lo_fin, _ = jax.lax.fori_loop(0, n_iters, body, (lo0, hi0))
    t_lo = jax.lax.bitcast_convert_type(lo_fin, jnp.float32)
    v = jnp.clip(g - (m + jnp.log(s)), _NEG_CLIP, 0.0)
    out_ref[...] = jnp.where(e > t_lo, v, _NEG_CLIP)


@jax.jit
def kernel(log_x_recon, cf_log_x_recon):
    b_dim, k_dim, hw = log_x_recon.shape
    tile = hw
    for cand in (256, 128, 64, 32, 16, 8):
        if hw % cand == 0:
            tile = cand
            break
    lo_init = int(np.float32(np.exp(np.float32(-11.0))).view(np.int32)) - 1
    hi_init = int(np.float32(1.0).view(np.int32))
    n_iters = max(1, (hi_init - lo_init).bit_length())

    grid = (b_dim, hw // tile)
    blk = pl.BlockSpec((1, k_dim, tile), lambda b, t: (b, 0, t))
    return pl.pallas_call(
        functools.partial(_topr_kernel, n_iters=n_iters,
                          lo_init=lo_init, hi_init=hi_init),
        grid=grid,
        in_specs=[blk, blk],
        out_specs=blk,
        out_shape=jax.ShapeDtypeStruct(log_x_recon.shape, log_x_recon.dtype),
    )(log_x_recon, cf_log_x_recon)
